# baseline (speedup 1.0000x reference)
import jax
import jax.numpy as jnp
from jax.experimental import pallas as pl
from jax.experimental.pallas import tpu as pltpu

BSZ = 4
SEQLEN = 4096
DIM = 1024
RATIO = 4
HEAD_DIM = 128
COFF = 2
RD = 64
ENTRIES_PER_BLOCK = 64
NUM_BLOCKS = 2048
MAX_BLOCKS = 16
EPS = 1e-6
C = COFF * HEAD_DIM
TOK = ENTRIES_PER_BLOCK * RATIO


def _body(phys_ref, x_ref, cosf_ref, sinf_ref, w_ref, apet_ref, nw_ref,
          sel_ref, cache_ref, out_ref):
    del phys_ref, cache_ref
    xb = x_ref[0].astype(jnp.bfloat16)
    y = jax.lax.dot_general(xb, w_ref[...], (((1,), (0,)), ((), ())),
                            preferred_element_type=jnp.float32)
    kv_lo = y[:, :128] * cosf_ref[...] + y[:, 512:640] * sinf_ref[...]
    kv = jnp.concatenate([kv_lo, y[:, 128:C]], axis=1)
    e = jnp.exp(y[:, C:2 * C])
    t = e * (kv + apet_ref[...])
    cat = jnp.concatenate([t, e], axis=1)
    nd = jax.lax.dot_general(sel_ref[...], cat, (((1,), (0,)), ((), ())),
                             preferred_element_type=jnp.float32)
    comp = nd[:, :C] / nd[:, C:]
    c0 = comp[:, :HEAD_DIM]
    c1 = comp[:, HEAD_DIM:]
    n0 = c0 * jax.lax.rsqrt(jnp.mean(c0 * c0, axis=1, keepdims=True) + EPS)
    n1 = c1 * jax.lax.rsqrt(jnp.mean(c1 * c1, axis=1, keepdims=True) + EPS)
    nw = nw_ref[...]
    out_ref[0] = jnp.concatenate([n0 * nw, n1 * nw], axis=1)


def kernel(x, start_pos, slot, freqs_cis, cache, block_offsets,
           Wkv, Wgate, ape, norm_w):
    del slot
    f32 = jnp.float32
    rot = Wkv[:RD].reshape(RD // 2, 2, DIM)
    wswap = jnp.stack([-rot[:, 1], rot[:, 0]], axis=1).reshape(RD, DIM)
    wswap = jnp.concatenate([wswap, jnp.zeros((128 - RD, DIM), f32)], axis=0)
    wcat = jnp.concatenate([Wkv, Wgate, wswap],
                           axis=0).T.astype(jnp.bfloat16)
    cosv = jnp.cos(freqs_cis)
    sinv = jnp.sin(freqs_cis)
    cosf = jnp.concatenate(
        [jnp.repeat(cosv, 2, axis=1), jnp.ones((SEQLEN, 128 - RD), f32)],
        axis=1)
    sinf = jnp.concatenate(
        [jnp.repeat(sinv, 2, axis=1), jnp.zeros((SEQLEN, 128 - RD), f32)],
        axis=1)
    apet = jnp.tile(ape, (ENTRIES_PER_BLOCK, 1))
    sel = (jnp.arange(TOK, dtype=jnp.int32)[None, :] // RATIO ==
           jnp.arange(ENTRIES_PER_BLOCK, dtype=jnp.int32)[:, None]
           ).astype(f32)
    lb = jnp.arange(MAX_BLOCKS, dtype=jnp.int32)[None, :]
    blk = start_pos[:, None] // (RATIO * ENTRIES_PER_BLOCK) + lb
    phys = block_offsets[jnp.arange(BSZ, dtype=jnp.int32)[:, None],
                         jnp.clip(blk, 0, block_offsets.shape[1] - 1)]

    grid_spec = pltpu.PrefetchScalarGridSpec(
        num_scalar_prefetch=1,
        grid=(BSZ, MAX_BLOCKS),
        in_specs=[
            pl.BlockSpec((1, TOK, DIM), lambda b, l, p: (b, l, 0)),
            pl.BlockSpec((TOK, 128), lambda b, l, p: (l, 0)),
            pl.BlockSpec((TOK, 128), lambda b, l, p: (l, 0)),
            pl.BlockSpec((DIM, 640), lambda b, l, p: (0, 0)),
            pl.BlockSpec((TOK, C), lambda b, l, p: (0, 0)),
            pl.BlockSpec((1, HEAD_DIM), lambda b, l, p: (0, 0)),
            pl.BlockSpec((ENTRIES_PER_BLOCK, TOK), lambda b, l, p: (0, 0)),
            pl.BlockSpec(memory_space=pl.ANY),
        ],
        out_specs=pl.BlockSpec((1, ENTRIES_PER_BLOCK, C),
                               lambda b, l, p: (p[b, l], 0, 0)),
    )
    return pl.pallas_call(
        _body,
        grid_spec=grid_spec,
        out_shape=jax.ShapeDtypeStruct(cache.shape, cache.dtype),
        input_output_aliases={8: 0},
        compiler_params=pltpu.CompilerParams(
            dimension_semantics=("arbitrary", "arbitrary")),
    )(phys, x, cosf, sinf, wcat, apet, norm_w.reshape(1, HEAD_DIM), sel,
      cache)
